# Initial kernel scaffold; baseline (speedup 1.0000x reference)
import jax
import jax.numpy as jnp
from jax.experimental import pallas as pl


def kernel(z, codebook):
    raise NotImplementedError("write your pallas kernel here")


# speedup vs baseline: 2.5875x; 2.5875x over previous
import functools

import jax
import jax.numpy as jnp
from jax import lax
from jax.experimental import pallas as pl

_CODEBOOK_SIZE = 1024
_COMMIT_W = 0.25


def _vq_body(z_ref, cb_ref, zq_ref, idx_ref, acc_ref):
    first = jnp.logical_and(pl.program_id(0) == 0, pl.program_id(1) == 0)

    @pl.when(first)
    def _():
        acc_ref[...] = jnp.zeros((1, 1), jnp.float32)

    zt = z_ref[0]
    cb = cb_ref[...]
    t = zt.shape[1]
    k = cb.shape[0]

    scores = lax.dot_general(cb, zt, (((1,), (0,)), ((), ())),
                             preferred_element_type=jnp.float32)
    cbsq = jnp.sum(cb * cb, axis=1, keepdims=True)
    ztsq = jnp.sum(zt * zt, axis=0, keepdims=True)
    dist = cbsq - 2.0 * scores + ztsq

    m = jnp.min(dist, axis=0, keepdims=True)
    kiota = lax.broadcasted_iota(jnp.int32, (k, t), 0)
    idx = jnp.min(jnp.where(dist == m, kiota, k), axis=0)

    onehot = (kiota == idx[None, :]).astype(jnp.float32)
    qcm = lax.dot_general(cb, onehot, (((0,), (0,)), ((), ())),
                          preferred_element_type=jnp.float32)

    err = qcm - zt
    acc_ref[...] += jnp.sum(err * err).reshape(1, 1)
    zq_ref[0] = qcm
    idx_ref[0, 0] = idx


@jax.jit
def kernel(z, codebook):
    b, c, f, h, w = z.shape
    n = f * h * w
    zc = z.reshape(b, c, n)
    tile = 1024
    n_t = n // tile
    k = codebook.shape[0]

    zq_cm, idx_arr, acc = pl.pallas_call(
        _vq_body,
        grid=(b, n_t),
        in_specs=[
            pl.BlockSpec((1, c, tile), lambda i, j: (i, 0, j)),
            pl.BlockSpec((k, c), lambda i, j: (0, 0)),
        ],
        out_specs=[
            pl.BlockSpec((1, c, tile), lambda i, j: (i, 0, j)),
            pl.BlockSpec((1, 1, tile), lambda i, j: (i * (n // tile) + j, 0, 0)),
            pl.BlockSpec((1, 1), lambda i, j: (0, 0)),
        ],
        out_shape=[
            jax.ShapeDtypeStruct((b, c, n), jnp.float32),
            jax.ShapeDtypeStruct((b * n_t, 1, tile), jnp.int32),
            jax.ShapeDtypeStruct((1, 1), jnp.float32),
        ],
    )(zc, codebook)

    zq = zq_cm.reshape(b, c, f, h, w)
    commit_loss = acc[0, 0] * (_COMMIT_W / (b * n * c))
    min_encoding_indices = idx_arr.reshape(-1, 1)
    return (zq, commit_loss, min_encoding_indices)


# speedup vs baseline: 3.2867x; 1.2702x over previous
import functools

import jax
import jax.numpy as jnp
from jax import lax
from jax.experimental import pallas as pl

_CODEBOOK_SIZE = 1024
_COMMIT_W = 0.25


def _vq_body(z_ref, cb_ref, zq_ref, idx_ref, acc_ref):
    first = jnp.logical_and(pl.program_id(0) == 0, pl.program_id(1) == 0)

    @pl.when(first)
    def _():
        acc_ref[...] = jnp.zeros((1, 1), jnp.float32)

    zt = z_ref[0]
    cb = cb_ref[...]
    t = zt.shape[1]
    k = cb.shape[0]

    scores2 = lax.dot_general(cb, -2.0 * zt, (((1,), (0,)), ((), ())),
                              preferred_element_type=jnp.float32)
    cbsq = jnp.sum(cb * cb, axis=1, keepdims=True)
    dist = scores2 + cbsq

    m = jnp.min(dist, axis=0, keepdims=True)
    ind = (dist == m).astype(jnp.float32)

    kcol = lax.broadcasted_iota(jnp.int32, (k, 1), 0)
    khi = (kcol // 32).astype(jnp.float32)
    klo = (kcol % 32).astype(jnp.float32)
    cbaug = jnp.concatenate([cb, khi, klo], axis=1)
    qa = lax.dot_general(cbaug, ind, (((0,), (0,)), ((), ())),
                         preferred_element_type=jnp.float32)
    qcm = qa[:-2, :]
    idx = (qa[-2:-1, :] * 32.0 + qa[-1:, :]).astype(jnp.int32)

    err = qcm - zt
    acc_ref[...] += jnp.sum(err * err).reshape(1, 1)
    zq_ref[0] = qcm
    idx_ref[0] = idx


@jax.jit
def kernel(z, codebook):
    b, c, f, h, w = z.shape
    n = f * h * w
    zc = z.reshape(b, c, n)
    tile = 1024
    n_t = n // tile
    k = codebook.shape[0]

    zq_cm, idx_arr, acc = pl.pallas_call(
        _vq_body,
        grid=(b, n_t),
        in_specs=[
            pl.BlockSpec((1, c, tile), lambda i, j: (i, 0, j)),
            pl.BlockSpec((k, c), lambda i, j: (0, 0)),
        ],
        out_specs=[
            pl.BlockSpec((1, c, tile), lambda i, j: (i, 0, j)),
            pl.BlockSpec((1, 1, tile), lambda i, j: (i * (n // tile) + j, 0, 0)),
            pl.BlockSpec((1, 1), lambda i, j: (0, 0)),
        ],
        out_shape=[
            jax.ShapeDtypeStruct((b, c, n), jnp.float32),
            jax.ShapeDtypeStruct((b * n_t, 1, tile), jnp.int32),
            jax.ShapeDtypeStruct((1, 1), jnp.float32),
        ],
    )(zc, codebook)

    zq = zq_cm.reshape(b, c, f, h, w)
    commit_loss = acc[0, 0] * (_COMMIT_W / (b * n * c))
    min_encoding_indices = idx_arr.reshape(-1, 1)
    return (zq, commit_loss, min_encoding_indices)


# speedup vs baseline: 3.8125x; 1.1600x over previous
import functools

import jax
import jax.numpy as jnp
from jax import lax
from jax.experimental import pallas as pl

_CODEBOOK_SIZE = 1024
_COMMIT_W = 0.25


def _vq_body(z_ref, cb_ref, zq_ref, idx_ref, acc_ref):
    first = jnp.logical_and(pl.program_id(0) == 0, pl.program_id(1) == 0)

    @pl.when(first)
    def _():
        acc_ref[...] = jnp.zeros((1, 1), jnp.float32)

    zt = z_ref[0]
    cb = cb_ref[...]
    t = zt.shape[1]
    k = cb.shape[0]

    scores2 = lax.dot_general(cb, -2.0 * zt, (((1,), (0,)), ((), ())),
                              preferred_element_type=jnp.float32)
    cbsq = jnp.sum(cb * cb, axis=1, keepdims=True)
    dist = scores2 + cbsq

    m = jnp.min(dist, axis=0, keepdims=True)
    ind = (dist == m).astype(jnp.float32)

    kcol = lax.broadcasted_iota(jnp.int32, (k, 1), 0)
    khi = (kcol // 32).astype(jnp.float32)
    klo = (kcol % 32).astype(jnp.float32)
    cbaug = jnp.concatenate([cb, khi, klo], axis=1)
    qa = lax.dot_general(cbaug, ind, (((0,), (0,)), ((), ())),
                         preferred_element_type=jnp.float32)
    qcm = qa[:-2, :]
    idx = (qa[-2:-1, :] * 32.0 + qa[-1:, :]).astype(jnp.int32)

    err = qcm - zt
    acc_ref[...] += jnp.sum(err * err).reshape(1, 1)
    zq_ref[0] = qcm
    idx_ref[0] = idx


@jax.jit
def kernel(z, codebook):
    b, c, f, h, w = z.shape
    n = f * h * w
    zc = z.reshape(b, c, n)
    tile = 4096
    n_t = n // tile
    k = codebook.shape[0]

    zq_cm, idx_arr, acc = pl.pallas_call(
        _vq_body,
        grid=(b, n_t),
        in_specs=[
            pl.BlockSpec((1, c, tile), lambda i, j: (i, 0, j)),
            pl.BlockSpec((k, c), lambda i, j: (0, 0)),
        ],
        out_specs=[
            pl.BlockSpec((1, c, tile), lambda i, j: (i, 0, j)),
            pl.BlockSpec((1, 1, tile), lambda i, j: (i * (n // tile) + j, 0, 0)),
            pl.BlockSpec((1, 1), lambda i, j: (0, 0)),
        ],
        out_shape=[
            jax.ShapeDtypeStruct((b, c, n), jnp.float32),
            jax.ShapeDtypeStruct((b * n_t, 1, tile), jnp.int32),
            jax.ShapeDtypeStruct((1, 1), jnp.float32),
        ],
    )(zc, codebook)

    zq = zq_cm.reshape(b, c, f, h, w)
    commit_loss = acc[0, 0] * (_COMMIT_W / (b * n * c))
    min_encoding_indices = idx_arr.reshape(-1, 1)
    return (zq, commit_loss, min_encoding_indices)
